# baseline (speedup 1.0000x reference)
import jax
import jax.numpy as jnp
from jax import lax
from jax.experimental import pallas as pl
from jax.experimental.pallas import tpu as pltpu
from jax.experimental.pallas import tpu_sc as plsc

U_N = 60000
I_N = 40000
N = U_N + I_N
D = 32
HD = D // 2
L_N = 3
E = 1600000

NC = 2
NS = 16
LANES = 16

CHUNK = 512
SUB = 128
NSUB = CHUNK // SUB
NCHUNK = 201
PER_S = CHUNK * NCHUNK
E_PAD = PER_S * NS
NPLANE = NS * NCHUNK


def _gcn_body(emb, packedc, packedr, packedv, o1, o2, o3, acc, cbuf, rbuf,
              vbuf, sidx, rowsbuf, sem_e, sem_g0, sem_g1, sem_g2, sem_s):
    c = lax.axis_index("c")
    s = lax.axis_index("s")
    zero16 = jnp.zeros((LANES,), jnp.float32)

    def zrow(i, carry):
        rowsbuf[0, i, 0:16] = zero16
        return carry

    for emb_l, out_l in ((emb, o1), (o1, o2), (o2, o3)):
        _one_layer(emb_l, packedc, packedr, packedv, out_l, acc, cbuf, rbuf,
                   vbuf, sidx, rowsbuf, sem_e, sem_g0, sem_g1, sem_g2, sem_s,
                   c, s, zrow)


def _one_layer(emb, packedc, packedr, packedv, out, acc, cbuf, rbuf, vbuf,
               sidx, rowsbuf, sem_e, sem_g0, sem_g1, sem_g2, sem_s,
               c, s, zrow):
    zero16 = jnp.zeros((LANES,), jnp.float32)

    lax.fori_loop(0, CHUNK, zrow, 0)
    zbase = s * 6250
    for i in range(12):
        pltpu.sync_copy(rowsbuf.at[0, pl.ds(0, 512)],
                        acc.at[pl.ds(zbase + i * 512, 512)])
    pltpu.sync_copy(rowsbuf.at[0, pl.ds(0, 106)],
                    acc.at[pl.ds(zbase + 12 * 512, 106)])
    plsc.subcore_barrier()

    def plane(ci):
        return s * NCHUNK + ci

    def fire_edge(ci, p):
        pltpu.async_copy(packedc.at[plane(ci)], cbuf.at[p], sem_e)
        pltpu.async_copy(packedr.at[plane(ci)], rbuf.at[p], sem_e)
        pltpu.async_copy(packedv.at[plane(ci)], vbuf.at[p], sem_e)

    def wait_edge(ci, p):
        pltpu.make_async_copy(packedc.at[plane(ci)], cbuf.at[p], sem_e).wait()
        pltpu.make_async_copy(packedr.at[plane(ci)], rbuf.at[p], sem_e).wait()
        pltpu.make_async_copy(packedv.at[plane(ci)], vbuf.at[p], sem_e).wait()

    sem_gs = (sem_g0, sem_g1, sem_g2)

    def fire_gathers(ci, p):
        for j in range(2):
            pltpu.async_copy(
                emb.at[c].at[cbuf.at[p, j]],
                rowsbuf.at[p, pl.ds(j * 256, 256)], sem_gs[p])

    def wait_gathers(p):
        for j in range(2):
            pltpu.make_async_copy(
                emb.at[c].at[cbuf.at[p, j]],
                rowsbuf.at[p, pl.ds(j * 256, 256)], sem_gs[p]).wait()

    def fire_scatters(p):
        for j in range(NSUB):
            pltpu.async_copy(
                rowsbuf.at[p, pl.ds(j * SUB, SUB)],
                acc.at[sidx.at[p, j]], sem_s, add=True)

    def wait_scatters(p):
        for j in range(NSUB):
            pltpu.make_async_copy(
                rowsbuf.at[p, pl.ds(j * SUB, SUB)],
                acc.at[sidx.at[p, j]], sem_s).wait()

    fire_edge(0, 0)
    wait_edge(0, 0)
    fire_gathers(0, 0)
    fire_edge(1, 1)
    wait_edge(1, 1)
    fire_gathers(1, 1)
    fire_edge(2, 2)

    def do_chunk(ci, p):
        pn = (p + 2) % 3
        wait_gathers(p)

        @pl.when(ci > 0)
        def _():
            wait_scatters(pn)

        @pl.when(ci + 2 < NCHUNK)
        def _():
            wait_edge(ci + 2, pn)
            fire_gathers(ci + 2, pn)

        for j in range(NSUB):
            def grp(g, carry2, j=j):
                kk = g * LANES
                sidx[p, j, pl.ds(kk, LANES)] = \
                    rbuf[p, j, pl.ds(kk, LANES)]
                val16 = vbuf[p, j, pl.ds(kk, LANES)]
                for t in range(LANES):
                    e = j * SUB + kk + t
                    vv = jnp.full((LANES,), val16[t], jnp.float32)
                    rowsbuf[p, e, 0:16] = rowsbuf[p, e, 0:16] * vv
                return carry2

            lax.fori_loop(0, SUB // LANES, grp, 0)

        fire_scatters(p)

        @pl.when(ci + 3 < NCHUNK)
        def _():
            fire_edge(ci + 3, p)

    def tri_body(k, carry):
        for r in range(3):
            do_chunk(3 * k + r, r)
        return carry

    lax.fori_loop(0, NCHUNK // 3, tri_body, 0)
    wait_scatters((NCHUNK - 1) % 3)
    plsc.subcore_barrier()

    SEG = 400
    for k in range(16):
        seg_id = s + k * NS

        @pl.when(seg_id < N // SEG)
        def _():
            sl = pl.ds(seg_id * SEG, SEG)
            pltpu.sync_copy(acc.at[sl], out.at[c].at[sl])
    plsc.subcore_barrier()


_gcn = pl.kernel(
    _gcn_body,
    out_type=(jax.ShapeDtypeStruct((NC, N, HD), jnp.float32),
              jax.ShapeDtypeStruct((NC, N, HD), jnp.float32),
              jax.ShapeDtypeStruct((NC, N, HD), jnp.float32)),
    mesh=plsc.VectorSubcoreMesh(core_axis_name="c", subcore_axis_name="s"),
    compiler_params=pltpu.CompilerParams(use_tc_tiling_on_sc=False),
    scratch_types=[
        pltpu.VMEM_SHARED((N, HD), jnp.float32),
        pltpu.VMEM((3, 2, 256), jnp.int32),
        pltpu.VMEM((3, NSUB, SUB), jnp.int32),
        pltpu.VMEM((3, NSUB, SUB), jnp.float32),
        pltpu.VMEM((3, NSUB, SUB), jnp.int32),
        pltpu.VMEM((3, CHUNK, HD), jnp.float32),
        pltpu.SemaphoreType.DMA,
        pltpu.SemaphoreType.DMA,
        pltpu.SemaphoreType.DMA,
        pltpu.SemaphoreType.DMA,
        pltpu.SemaphoreType.DMA,
    ],
)


def _mean4_body(a, b, c, d, o):
    bb = jnp.concatenate([b[0], b[1]], axis=-1)
    cc = jnp.concatenate([c[0], c[1]], axis=-1)
    dd = jnp.concatenate([d[0], d[1]], axis=-1)
    o[...] = (a[...] + bb + cc + dd) * jnp.float32(0.25)


def _mean4(e0, s1, s2, s3):
    blk = 5000
    grid = (N // blk,)
    spec_full = pl.BlockSpec((blk, D), lambda i: (i, 0))
    spec_half = pl.BlockSpec((NC, blk, HD), lambda i: (0, i, 0))
    return pl.pallas_call(
        _mean4_body,
        grid=grid,
        in_specs=[spec_full, spec_half, spec_half, spec_half],
        out_specs=spec_full,
        out_shape=jax.ShapeDtypeStruct((N, D), jnp.float32),
    )(e0, s1, s2, s3)


def kernel(edge_index, edge_values, user_emb, item_emb):
    emb0 = jnp.concatenate([user_emb, item_emb], axis=0)
    row = edge_index[0]
    col = edge_index[1]
    pad = E_PAD - E
    spread = (jnp.arange(pad, dtype=jnp.int32) * 97) % N
    packedc = jnp.concatenate([col, spread]).reshape(NPLANE, 2, 256)
    packedr = jnp.concatenate([row, spread]).reshape(NPLANE, NSUB, SUB)
    packedv = jnp.pad(edge_values, (0, pad)).reshape(NPLANE, NSUB, SUB)

    cur = jnp.stack([emb0[:, :HD], emb0[:, HD:]], axis=0)
    s1, s2, s3 = _gcn(cur, packedc, packedr, packedv)
    out = _mean4(emb0, s1, s2, s3)
    return (emb0, out)


# speedup vs baseline: 1.0685x; 1.0685x over previous
import jax
import jax.numpy as jnp
from jax import lax
from jax.experimental import pallas as pl
from jax.experimental.pallas import tpu as pltpu
from jax.experimental.pallas import tpu_sc as plsc

U_N = 60000
I_N = 40000
N = U_N + I_N
D = 32
HD = D // 2
L_N = 3
E = 1600000

NC = 2
NS = 16
LANES = 16

CHUNK = 512
SUB = 128
NSUB = CHUNK // SUB
NCHUNK = 201
PER_S = CHUNK * NCHUNK
E_PAD = PER_S * NS
NPLANE = NS * NCHUNK


def _layer_body(emb, packedc, packedr, packedv, out, acc, cbuf, rbuf, vbuf,
                sidx, rowsbuf, sem_e, sem_g0, sem_g1, sem_g2, sem_s):
    c = lax.axis_index("c")
    s = lax.axis_index("s")

    zero16 = jnp.zeros((LANES,), jnp.float32)

    def zrow(i, carry):
        rowsbuf[0, i, 0:16] = zero16
        return carry

    lax.fori_loop(0, CHUNK, zrow, 0)
    zbase = s * 6250
    for i in range(12):
        pltpu.sync_copy(rowsbuf.at[0, pl.ds(0, 512)],
                        acc.at[pl.ds(zbase + i * 512, 512)])
    pltpu.sync_copy(rowsbuf.at[0, pl.ds(0, 106)],
                    acc.at[pl.ds(zbase + 12 * 512, 106)])
    plsc.subcore_barrier()

    def plane(ci):
        return s * NCHUNK + ci

    def fire_edge(ci, p):
        pltpu.async_copy(packedc.at[plane(ci)], cbuf.at[p], sem_e)
        pltpu.async_copy(packedr.at[plane(ci)], rbuf.at[p], sem_e)
        pltpu.async_copy(packedv.at[plane(ci)], vbuf.at[p], sem_e)

    def wait_edge(ci, p):
        pltpu.make_async_copy(packedc.at[plane(ci)], cbuf.at[p], sem_e).wait()
        pltpu.make_async_copy(packedr.at[plane(ci)], rbuf.at[p], sem_e).wait()
        pltpu.make_async_copy(packedv.at[plane(ci)], vbuf.at[p], sem_e).wait()

    sem_gs = (sem_g0, sem_g1, sem_g2)

    def fire_gathers(ci, p):
        for j in range(2):
            pltpu.async_copy(
                emb.at[c].at[cbuf.at[p, j]],
                rowsbuf.at[p, pl.ds(j * 256, 256)], sem_gs[p])

    def wait_gathers(p):
        for j in range(2):
            pltpu.make_async_copy(
                emb.at[c].at[cbuf.at[p, j]],
                rowsbuf.at[p, pl.ds(j * 256, 256)], sem_gs[p]).wait()

    def fire_scatters(p):
        for j in range(2):
            pltpu.async_copy(
                rowsbuf.at[p, pl.ds(j * 256, 256)],
                acc.at[sidx.at[p, j]], sem_s, add=True)

    def wait_scatters(p):
        for j in range(2):
            pltpu.make_async_copy(
                rowsbuf.at[p, pl.ds(j * 256, 256)],
                acc.at[sidx.at[p, j]], sem_s).wait()

    fire_edge(0, 0)
    wait_edge(0, 0)
    fire_gathers(0, 0)
    fire_edge(1, 1)
    wait_edge(1, 1)
    fire_gathers(1, 1)
    fire_edge(2, 2)

    def do_chunk(ci, p):
        pn = (p + 2) % 3
        wait_gathers(p)

        @pl.when(ci > 0)
        def _():
            wait_scatters(pn)

        @pl.when(ci + 2 < NCHUNK)
        def _():
            wait_edge(ci + 2, pn)
            fire_gathers(ci + 2, pn)

        for j in range(NSUB):
            def grp(g, carry2, j=j):
                kk = g * LANES
                sidx[p, j // 2, pl.ds((j % 2) * SUB + kk, LANES)] = \
                    rbuf[p, j, pl.ds(kk, LANES)]
                val16 = vbuf[p, j, pl.ds(kk, LANES)]
                for t in range(LANES):
                    e = j * SUB + kk + t
                    vv = jnp.full((LANES,), val16[t], jnp.float32)
                    rowsbuf[p, e, 0:16] = rowsbuf[p, e, 0:16] * vv
                return carry2

            lax.fori_loop(0, SUB // LANES, grp, 0)

        fire_scatters(p)

        @pl.when(ci + 3 < NCHUNK)
        def _():
            fire_edge(ci + 3, p)

    def tri_body(k, carry):
        for r in range(3):
            do_chunk(3 * k + r, r)
        return carry

    lax.fori_loop(0, NCHUNK // 3, tri_body, 0)
    wait_scatters((NCHUNK - 1) % 3)
    plsc.subcore_barrier()

    SEG = 400
    for k in range(16):
        seg_id = s + k * NS

        @pl.when(seg_id < N // SEG)
        def _():
            sl = pl.ds(seg_id * SEG, SEG)
            pltpu.sync_copy(acc.at[sl], out.at[c].at[sl])


_layer = pl.kernel(
    _layer_body,
    out_type=jax.ShapeDtypeStruct((NC, N, HD), jnp.float32),
    mesh=plsc.VectorSubcoreMesh(core_axis_name="c", subcore_axis_name="s"),
    compiler_params=pltpu.CompilerParams(use_tc_tiling_on_sc=False),
    scratch_types=[
        pltpu.VMEM_SHARED((N, HD), jnp.float32),
        pltpu.VMEM((3, 2, 256), jnp.int32),
        pltpu.VMEM((3, NSUB, SUB), jnp.int32),
        pltpu.VMEM((3, NSUB, SUB), jnp.float32),
        pltpu.VMEM((3, 2, 256), jnp.int32),
        pltpu.VMEM((3, CHUNK, HD), jnp.float32),
        pltpu.SemaphoreType.DMA,
        pltpu.SemaphoreType.DMA,
        pltpu.SemaphoreType.DMA,
        pltpu.SemaphoreType.DMA,
        pltpu.SemaphoreType.DMA,
    ],
)


def _mean4_body(a, b, c, d, o):
    bb = jnp.concatenate([b[0], b[1]], axis=-1)
    cc = jnp.concatenate([c[0], c[1]], axis=-1)
    dd = jnp.concatenate([d[0], d[1]], axis=-1)
    o[...] = (a[...] + bb + cc + dd) * jnp.float32(0.25)


def _mean4(e0, s1, s2, s3):
    blk = 5000
    grid = (N // blk,)
    spec_full = pl.BlockSpec((blk, D), lambda i: (i, 0))
    spec_half = pl.BlockSpec((NC, blk, HD), lambda i: (0, i, 0))
    return pl.pallas_call(
        _mean4_body,
        grid=grid,
        in_specs=[spec_full, spec_half, spec_half, spec_half],
        out_specs=spec_full,
        out_shape=jax.ShapeDtypeStruct((N, D), jnp.float32),
    )(e0, s1, s2, s3)


def kernel(edge_index, edge_values, user_emb, item_emb):
    emb0 = jnp.concatenate([user_emb, item_emb], axis=0)
    row = edge_index[0]
    col = edge_index[1]
    pad = E_PAD - E
    spread = (jnp.arange(pad, dtype=jnp.int32) * 97) % N
    packedc = jnp.concatenate([col, spread]).reshape(NPLANE, 2, 256)
    packedr = jnp.concatenate([row, spread]).reshape(NPLANE, NSUB, SUB)
    packedv = jnp.pad(edge_values, (0, pad)).reshape(NPLANE, NSUB, SUB)

    cur = jnp.stack([emb0[:, :HD], emb0[:, HD:]], axis=0)
    stacked = []
    for _ in range(L_N):
        cur = _layer(cur, packedc, packedr, packedv)
        stacked.append(cur)
    out = _mean4(emb0, *stacked)
    return (emb0, out)
